# baseline (speedup 1.0000x reference)
import jax
import jax.numpy as jnp
from jax.experimental import pallas as pl

N_NODES = 10000
D_IN = 256
D_OUT = 256


def _tc_body(x_ref, agg_ref, deg_ref, ws_ref, wn_ref, o_ref):
    xb = x_ref[...]
    deg = jnp.maximum(deg_ref[:, 0:1], 1.0)
    nm = agg_ref[...] / deg
    h_self = jnp.dot(xb, ws_ref[...], preferred_element_type=jnp.float32)
    h_neigh = jnp.dot(nm, wn_ref[...], preferred_element_type=jnp.float32)
    o_ref[...] = jnp.maximum(jnp.concatenate([h_self, h_neigh], axis=1), 0.0)


def _tc_combine(x, agg, deg, W_self, W_neigh):
    BR = 400
    grid = (N_NODES // BR,)
    return pl.pallas_call(
        _tc_body,
        grid=grid,
        in_specs=[
            pl.BlockSpec((BR, D_IN), lambda i: (i, 0)),
            pl.BlockSpec((BR, D_IN), lambda i: (i, 0)),
            pl.BlockSpec((BR, 8), lambda i: (i, 0)),
            pl.BlockSpec((D_IN, D_OUT), lambda i: (0, 0)),
            pl.BlockSpec((D_IN, D_OUT), lambda i: (0, 0)),
        ],
        out_specs=pl.BlockSpec((BR, 2 * D_OUT), lambda i: (i, 0)),
        out_shape=jax.ShapeDtypeStruct((N_NODES, 2 * D_OUT), jnp.float32),
    )(x, agg, deg, W_self, W_neigh)


def kernel(x, edge_index, W_self, W_neigh):
    src = edge_index[0]
    dst = edge_index[1]
    msgs = jnp.take(x, src, axis=0)
    agg = jax.ops.segment_sum(msgs, dst, num_segments=N_NODES)
    deg = jax.ops.segment_sum(
        jnp.ones((src.shape[0],), dtype=x.dtype), dst, num_segments=N_NODES)
    deg8 = jnp.broadcast_to(deg[:, None], (N_NODES, 8))
    return _tc_combine(x, agg, deg8, W_self, W_neigh)


# speedup vs baseline: 1.0754x; 1.0754x over previous
import jax
import jax.numpy as jnp
from jax.experimental import pallas as pl

N_NODES = 10000
D_IN = 256
D_OUT = 256


def _tc_body(x_ref, agg_ref, deg_ref, ws_ref, wn_ref, o_ref):
    xb = x_ref[...]
    deg = jnp.maximum(deg_ref[:, 0:1], 1.0)
    nm = agg_ref[...] / deg
    h_self = jnp.dot(xb, ws_ref[...], preferred_element_type=jnp.float32)
    h_neigh = jnp.dot(nm, wn_ref[...], preferred_element_type=jnp.float32)
    o_ref[...] = jnp.maximum(jnp.concatenate([h_self, h_neigh], axis=1), 0.0)


def _tc_combine(x, agg, deg, W_self, W_neigh):
    BR = 400
    grid = (N_NODES // BR,)
    return pl.pallas_call(
        _tc_body,
        grid=grid,
        in_specs=[
            pl.BlockSpec((BR, D_IN), lambda i: (i, 0)),
            pl.BlockSpec((BR, D_IN), lambda i: (i, 0)),
            pl.BlockSpec((BR, 8), lambda i: (i, 0)),
            pl.BlockSpec((D_IN, D_OUT), lambda i: (0, 0)),
            pl.BlockSpec((D_IN, D_OUT), lambda i: (0, 0)),
        ],
        out_specs=pl.BlockSpec((BR, 2 * D_OUT), lambda i: (i, 0)),
        out_shape=jax.ShapeDtypeStruct((N_NODES, 2 * D_OUT), jnp.float32),
    )(x, agg, deg, W_self, W_neigh)


def kernel(x, edge_index, W_self, W_neigh):
    src = edge_index[0]
    dst = edge_index[1]
    msgs = jnp.take(x, src, axis=0, mode="clip")
    agg = jax.ops.segment_sum(msgs, dst, num_segments=N_NODES)
    deg = jax.ops.segment_sum(
        jnp.ones((src.shape[0],), dtype=x.dtype), dst, num_segments=N_NODES)
    deg8 = jnp.broadcast_to(deg[:, None], (N_NODES, 8))
    return _tc_combine(x, agg, deg8, W_self, W_neigh)
